# baseline (speedup 1.0000x reference)
import functools

import jax
import jax.numpy as jnp
from jax import lax
from jax.experimental import pallas as pl
from jax.experimental.pallas import tpu as pltpu
from jax.experimental.pallas import tpu_sc as plsc

ROWS = 32768
CH = 2048
L = 16
NC = 2
NS = 16
NW = NC * NS
ROWS_PER_W = ROWS // NW
R = 8
NCHUNK = ROWS_PER_W // R
NQUAD = NCHUNK // 4
G = CH // L
CB = R * CH

_mesh = plsc.VectorSubcoreMesh(core_axis_name="c", subcore_axis_name="s")


@functools.partial(
    pl.kernel,
    mesh=_mesh,
    out_type=jax.ShapeDtypeStruct((ROWS * CH,), jnp.float32),
    scratch_types=[
        pltpu.VMEM((CH,), jnp.int32),
        pltpu.VMEM((CB,), jnp.float32),
        pltpu.VMEM((CB,), jnp.float32),
        pltpu.VMEM((CB,), jnp.float32),
        pltpu.VMEM((CB,), jnp.float32),
        pltpu.VMEM((CB,), jnp.float32),
        pltpu.VMEM((CB,), jnp.float32),
        pltpu.SemaphoreType.DMA,
        pltpu.SemaphoreType.DMA,
        pltpu.SemaphoreType.DMA,
        pltpu.SemaphoreType.DMA,
        pltpu.SemaphoreType.DMA,
        pltpu.SemaphoreType.DMA,
    ],
    compiler_params=pltpu.CompilerParams(needs_layout_passes=False),
)
def _permute(in_hbm, pidx_hbm, out_hbm, pidx_v,
             in0, in1, out0, out1, out2, out3,
             si0, si1, so0, so1, so2, so3):
    wid = lax.axis_index("s") * NC + lax.axis_index("c")
    base = wid * ROWS_PER_W * CH
    pltpu.sync_copy(pidx_hbm, pidx_v)

    ins, outs = [in0, in1], [out0, out1, out2, out3]
    sis, sos = [si0, si1], [so0, so1, so2, so3]

    def in_copy(n, b):
        return pltpu.make_async_copy(
            in_hbm.at[pl.ds(base + n * CB, CB)], ins[b], sis[b])

    def out_copy(n, b):
        return pltpu.make_async_copy(
            outs[b], out_hbm.at[pl.ds(base + n * CB, CB)], sos[b])

    def compute(bi, bo):
        inbuf, outbuf = ins[bi], outs[bo]

        @plsc.parallel_loop(0, G, unroll=2)
        def _group_body(g):
            q0 = (g >> 3) * 1024 + (g & 7) * L
            idx = pidx_v[pl.ds(g * L, L)]
            for r in range(R):
                v = plsc.load_gather(inbuf, [idx + (r * 128)])
                outbuf[pl.ds(q0 + r * 128, L)] = v

    in_copy(0, 0).start()

    def quad_body(c, carry):
        n0 = 4 * c
        for b in range(4):
            n = n0 + b
            if b < 3:
                in_copy(n + 1, (b + 1) % 2).start()
            else:
                @pl.when(c < NQUAD - 1)
                def _():
                    in_copy(n + 1, (b + 1) % 2).start()

            in_copy(n, b % 2).wait()

            @pl.when(c > 0)
            def _():
                out_copy(n - 4, b).wait()

            compute(b % 2, b)
            out_copy(n, b).start()
        return carry

    lax.fori_loop(0, NQUAD, quad_body, 0)
    for b in range(4):
        out_copy(NCHUNK - 4 + b, b).wait()


def kernel(input, perm):
    perm = perm.astype(jnp.int32)
    pidx = (perm // 128) * 1024 + (perm % 128)
    in_phys = input.reshape(ROWS // 8, 8, CH // 128, 128)
    in_phys = in_phys.transpose(0, 2, 1, 3).reshape(-1)
    out_phys = _permute(in_phys, pidx)
    out = out_phys.reshape(ROWS // 8, CH // 128, 8, 128)
    return out.transpose(0, 2, 1, 3).reshape(ROWS, CH)


# speedup vs baseline: 1.0178x; 1.0178x over previous
import functools

import jax
import jax.numpy as jnp
from jax import lax
from jax.experimental import pallas as pl
from jax.experimental.pallas import tpu as pltpu
from jax.experimental.pallas import tpu_sc as plsc

ROWS = 32768
CH = 2048
L = 16
NC = 2
NS = 16
NW = NC * NS
ROWS_PER_W = ROWS // NW
RI = 16
RO = 8
NSLAB = ROWS_PER_W // RI
G = CH // L
CBI = RI * CH
CBO = RO * CH

_mesh = plsc.VectorSubcoreMesh(core_axis_name="c", subcore_axis_name="s")


@functools.partial(
    pl.kernel,
    mesh=_mesh,
    out_type=jax.ShapeDtypeStruct((ROWS * CH,), jnp.float32),
    scratch_types=[
        pltpu.VMEM((CH,), jnp.int32),
        pltpu.VMEM((CBI,), jnp.float32),
        pltpu.VMEM((CBI,), jnp.float32),
        pltpu.VMEM((CBO,), jnp.float32),
        pltpu.VMEM((CBO,), jnp.float32),
        pltpu.SemaphoreType.DMA,
        pltpu.SemaphoreType.DMA,
        pltpu.SemaphoreType.DMA,
        pltpu.SemaphoreType.DMA,
    ],
    compiler_params=pltpu.CompilerParams(needs_layout_passes=False),
)
def _permute(in_hbm, pidx_hbm, out_hbm, pidx_v,
             in0, in1, out0, out1,
             si0, si1, so0, so1):
    wid = lax.axis_index("s") * NC + lax.axis_index("c")
    base = wid * ROWS_PER_W * CH
    pltpu.sync_copy(pidx_hbm, pidx_v)

    ins, outs = [in0, in1], [out0, out1]
    sis, sos = [si0, si1], [so0, so1]

    def in_copy(n, b):
        return pltpu.make_async_copy(
            in_hbm.at[pl.ds(base + n * CBI, CBI)], ins[b], sis[b])

    def out_copy(n, h):
        return pltpu.make_async_copy(
            outs[h], out_hbm.at[pl.ds(base + (2 * n + h) * CBO, CBO)], sos[h])

    def compute(bi, h):
        inbuf, outbuf = ins[bi], outs[h]
        hoff = h * CBO

        @plsc.parallel_loop(0, G, unroll=2)
        def _group_body(g):
            q0 = (g >> 3) * 1024 + (g & 7) * L
            idx = pidx_v[pl.ds(g * L, L)] + hoff
            for r in range(RO):
                v = plsc.load_gather(inbuf, [idx + (r * 128)])
                outbuf[pl.ds(q0 + r * 128, L)] = v

    in_copy(0, 0).start()
    NPAIR = NSLAB // 2

    def pair_body(c, carry):
        n0 = 2 * c
        for k in range(2):
            n = n0 + k
            if k == 0:
                in_copy(n + 1, 1).start()
            else:
                @pl.when(c < NPAIR - 1)
                def _():
                    in_copy(n + 1, 0).start()

            in_copy(n, k).wait()

            for h in range(2):
                @pl.when(n > 0)
                def _():
                    out_copy(n - 1, h).wait()

                compute(k, h)
                out_copy(n, h).start()
        return carry

    lax.fori_loop(0, NPAIR, pair_body, 0)
    for h in range(2):
        out_copy(NSLAB - 1, h).wait()


def kernel(input, perm):
    perm = perm.astype(jnp.int32)
    pidx = (perm // 128) * 1024 + (perm % 128)
    in_phys = input.reshape(ROWS // 8, 8, CH // 128, 128)
    in_phys = in_phys.transpose(0, 2, 1, 3).reshape(-1)
    out_phys = _permute(in_phys, pidx)
    out = out_phys.reshape(ROWS // 8, CH // 128, 8, 128)
    return out.transpose(0, 2, 1, 3).reshape(ROWS, CH)
